# baseline (speedup 1.0000x reference)
import functools

import jax
import jax.numpy as jnp
from jax import lax
from jax.experimental import pallas as pl
from jax.experimental.pallas import tpu as pltpu
from jax.experimental.pallas import tpu_sc as plsc

_VOCAB = 26
_ADIM = 128
_SEQ = 8192

_CHUNK = 128


def _make_sc_embed():
    info = plsc.get_sparse_core_info()
    num_cores = 1
    nw = num_cores * info.num_subcores
    b_per_w = _SEQ // nw
    n_chunks = b_per_w // _CHUNK
    mesh = plsc.VectorSubcoreMesh(
        core_axis_name="c", subcore_axis_name="s", num_cores=num_cores
    )

    @functools.partial(
        pl.kernel,
        mesh=mesh,
        out_type=jax.ShapeDtypeStruct((_SEQ // _CHUNK, _CHUNK, _ADIM), jnp.float32),
        scratch_types=[
            pltpu.VMEM((n_chunks, _CHUNK), jnp.int32),
            pltpu.VMEM_SHARED((_VOCAB, _ADIM), jnp.float32),
            pltpu.VMEM((n_chunks, _CHUNK, _ADIM), jnp.float32),
            pltpu.SemaphoreType.DMA,
        ],
    )
    def emb(idx_hbm, table_hbm, out_hbm, idx_v, table_sh, rows_v, gsem):
        sid = lax.axis_index("s")
        row0 = sid * n_chunks

        @pl.when(sid == 0)
        def _stage_table():
            pltpu.async_copy(table_hbm, table_sh, gsem)

        pltpu.sync_copy(idx_hbm.at[pl.ds(row0, n_chunks)], idx_v)

        @pl.when(sid == 0)
        def _wait_table():
            pltpu.make_async_copy(table_hbm, table_sh, gsem).wait()

        plsc.subcore_barrier()

        gathers = [
            pltpu.async_copy(table_sh.at[idx_v.at[j]], rows_v.at[j], gsem)
            for j in range(n_chunks)
        ]
        for g in gathers:
            g.wait()
        pltpu.sync_copy(rows_v, out_hbm.at[pl.ds(row0, n_chunks)])

    return emb


_sc_embed = _make_sc_embed()


def kernel(indices, table):
    idx2d = indices.reshape(_SEQ // _CHUNK, _CHUNK)
    out = _sc_embed(idx2d, table)
    return out.reshape(1, _SEQ, _ADIM)


# speedup vs baseline: 1.0576x; 1.0576x over previous
import functools

import jax
import jax.numpy as jnp
from jax import lax
from jax.experimental import pallas as pl
from jax.experimental.pallas import tpu as pltpu
from jax.experimental.pallas import tpu_sc as plsc

_VOCAB = 26
_ADIM = 128
_SEQ = 8192

_CHUNK = 128


def _make_sc_embed():
    info = plsc.get_sparse_core_info()
    num_cores = info.num_cores
    nw = num_cores * info.num_subcores
    b_per_w = _SEQ // nw
    n_chunks = b_per_w // _CHUNK
    mesh = plsc.VectorSubcoreMesh(
        core_axis_name="c", subcore_axis_name="s", num_cores=num_cores
    )

    @functools.partial(
        pl.kernel,
        mesh=mesh,
        out_type=jax.ShapeDtypeStruct((_SEQ // _CHUNK, _CHUNK, _ADIM), jnp.float32),
        scratch_types=[
            pltpu.VMEM((n_chunks, _CHUNK), jnp.int32),
            pltpu.VMEM_SHARED((_VOCAB, _ADIM), jnp.float32),
            pltpu.VMEM((n_chunks, _CHUNK, _ADIM), jnp.float32),
            pltpu.SemaphoreType.DMA,
            pltpu.SemaphoreType.DMA,
        ],
    )
    def emb(idx_hbm, table_hbm, out_hbm, idx_v, table_sh, rows_v, gsem, ssem):
        sid = lax.axis_index("s")
        wid = lax.axis_index("c") * info.num_subcores + sid
        row0 = wid * n_chunks

        @pl.when(sid == 0)
        def _stage_table():
            pltpu.async_copy(table_hbm, table_sh, gsem)

        pltpu.sync_copy(idx_hbm.at[pl.ds(row0, n_chunks)], idx_v)

        @pl.when(sid == 0)
        def _wait_table():
            pltpu.make_async_copy(table_hbm, table_sh, gsem).wait()

        plsc.subcore_barrier()

        gathers = [
            pltpu.async_copy(table_sh.at[idx_v.at[j]], rows_v.at[j], gsem)
            for j in range(n_chunks)
        ]
        for j in range(n_chunks):
            gathers[j].wait()
            pltpu.async_copy(rows_v.at[j], out_hbm.at[row0 + j], ssem)
        for j in range(n_chunks):
            pltpu.make_async_copy(rows_v.at[j], out_hbm.at[row0 + j], ssem).wait()

    return emb


_sc_embed = _make_sc_embed()


def kernel(indices, table):
    idx2d = indices.reshape(_SEQ // _CHUNK, _CHUNK)
    out = _sc_embed(idx2d, table)
    return out.reshape(1, _SEQ, _ADIM)
